# baseline (speedup 1.0000x reference)
import functools

import jax
import jax.numpy as jnp
from jax.experimental import pallas as pl
from jax.experimental.pallas import tpu as pltpu


def _mm_body(x_ref, w_ref, b_ref, o_ref):
    o_ref[...] = jnp.dot(x_ref[...], w_ref[...],
                         preferred_element_type=jnp.float32) + b_ref[...]


def _mm(x, w, b):
    n, k = x.shape
    m = w.shape[1]
    return pl.pallas_call(
        _mm_body,
        out_shape=jax.ShapeDtypeStruct((n, m), jnp.float32),
        in_specs=[
            pl.BlockSpec((1000, k), lambda i: (i, 0)),
            pl.BlockSpec((k, m), lambda i: (0, 0)),
            pl.BlockSpec((m,), lambda i: (0,)),
        ],
        out_specs=pl.BlockSpec((1000, m), lambda i: (i, 0)),
        grid=(n // 1000,),
    )(x, w, b)


def kernel(features, edge_index, W0, b0, W1, b1, Wfc, bfc):
    src = edge_index[0]
    dst = edge_index[1]
    N = features.shape[0]
    ones = jnp.ones((src.shape[0],), dtype=jnp.float32)
    deg_out = jax.ops.segment_sum(ones, src, num_segments=N)
    deg_in = jax.ops.segment_sum(ones, dst, num_segments=N)
    norm_out = jnp.where(deg_out > 0, deg_out ** -0.5, 0.0)
    norm_in = jnp.where(deg_in > 0, deg_in ** -0.5, 0.0)

    def gconv(h, W, b):
        h = _mm(h, W, jnp.zeros((W.shape[1],), jnp.float32))
        h = h * norm_out[:, None]
        agg = jax.ops.segment_sum(h[src], dst, num_segments=N)
        agg = agg * norm_in[:, None]
        return jax.nn.relu(agg + b)

    h = gconv(features, W0, b0)
    h = gconv(h, W1, b1)
    return _mm(h, Wfc, bfc)


# speedup vs baseline: 6.0958x; 6.0958x over previous
import functools

import jax
import jax.numpy as jnp
from jax import lax
from jax.experimental import pallas as pl
from jax.experimental.pallas import tpu as pltpu
from jax.experimental.pallas import tpu_sc as plsc

N_NODES = 10000
N_EDGES = 320000
FEA = 128
NC = 2
NS = 16
NW = NC * NS
EDGES_PER_TILE = N_EDGES // NW
CHUNK = 125
NCHUNK = EDGES_PER_TILE // CHUNK
ROWS_PER_TILE = N_NODES // NS

_mesh = plsc.VectorSubcoreMesh(
    core_axis_name="c", subcore_axis_name="s", num_cores=NC, num_subcores=NS)


@functools.partial(
    pl.kernel,
    out_type=[jax.ShapeDtypeStruct((NC, NS, ROWS_PER_TILE, FEA), jnp.float32),
              jax.ShapeDtypeStruct((NC, NS, ROWS_PER_TILE, FEA), jnp.float32)],
    mesh=_mesh,
    scratch_types=[
        pltpu.VMEM((NCHUNK, CHUNK), jnp.int32),
        pltpu.VMEM((NCHUNK, CHUNK), jnp.int32),
        pltpu.VMEM((CHUNK, FEA), jnp.float32),
        pltpu.VMEM_SHARED((N_NODES, FEA), jnp.float32),
    ],
)
def _deg_kernel(src_hbm, dst_hbm, ones_hbm, zeros_hbm,
                out_s, out_d, src_v, dst_v, ones_v, tab):
    c = lax.axis_index("c")
    s = lax.axis_index("s")
    wid = c * NS + s
    pltpu.sync_copy(src_hbm.at[wid], src_v)
    pltpu.sync_copy(dst_hbm.at[wid], dst_v)
    pltpu.sync_copy(ones_hbm, ones_v)
    for idx_v, out_hbm in ((src_v, out_s), (dst_v, out_d)):
        pltpu.sync_copy(zeros_hbm, tab.at[pl.ds(s * ROWS_PER_TILE, ROWS_PER_TILE)])
        plsc.subcore_barrier()

        def body(j, carry):
            pltpu.sync_copy(ones_v, tab.at[idx_v.at[j]], add=True)
            return carry

        lax.fori_loop(0, NCHUNK, body, 0)
        plsc.subcore_barrier()
        pltpu.sync_copy(tab.at[pl.ds(s * ROWS_PER_TILE, ROWS_PER_TILE)],
                        out_hbm.at[c, s])
        plsc.subcore_barrier()


@functools.partial(
    pl.kernel,
    out_type=jax.ShapeDtypeStruct((NC, NS, ROWS_PER_TILE, FEA), jnp.float32),
    mesh=_mesh,
    scratch_types=[
        pltpu.VMEM((NCHUNK, CHUNK), jnp.int32),
        pltpu.VMEM((NCHUNK, CHUNK), jnp.int32),
        pltpu.VMEM((CHUNK, FEA), jnp.float32),
        pltpu.VMEM_SHARED((N_NODES, FEA), jnp.float32),
        pltpu.SemaphoreType.DMA,
    ],
)
def _agg_kernel(ht_hbm, src_hbm, dst_hbm, zeros_hbm,
                out_hbm, src_v, dst_v, rows_v, accum, sem):
    c = lax.axis_index("c")
    s = lax.axis_index("s")
    wid = c * NS + s
    pltpu.sync_copy(zeros_hbm, accum.at[pl.ds(s * ROWS_PER_TILE, ROWS_PER_TILE)])
    pltpu.sync_copy(src_hbm.at[wid], src_v)
    pltpu.sync_copy(dst_hbm.at[wid], dst_v)
    plsc.subcore_barrier()

    def body(j, carry):
        pltpu.async_copy(ht_hbm.at[src_v.at[j]], rows_v, sem).wait()
        pltpu.sync_copy(rows_v, accum.at[dst_v.at[j]], add=True)
        return carry

    lax.fori_loop(0, NCHUNK, body, 0)
    plsc.subcore_barrier()
    pltpu.sync_copy(accum.at[pl.ds(s * ROWS_PER_TILE, ROWS_PER_TILE)],
                    out_hbm.at[c, s])


_BLK = 1000


def _norm_from_tab(tab):
    deg = tab[0, :, 0] + tab[1, :, 0]
    return jnp.where(deg > 0, lax.rsqrt(deg), 0.0)


def _tc1_body(tab_s_ref, x_ref, w_ref, o_ref):
    norm_out = _norm_from_tab(tab_s_ref[...])
    o_ref[...] = jnp.dot(x_ref[...], w_ref[...],
                         preferred_element_type=jnp.float32) * norm_out[:, None]


def _tc1(tab_s, x, w):
    n, k = x.shape
    return pl.pallas_call(
        _tc1_body,
        out_shape=jax.ShapeDtypeStruct((n, w.shape[1]), jnp.float32),
        in_specs=[
            pl.BlockSpec((NC, _BLK, FEA), lambda i: (0, i, 0)),
            pl.BlockSpec((_BLK, k), lambda i: (i, 0)),
            pl.BlockSpec((k, w.shape[1]), lambda i: (0, 0)),
        ],
        out_specs=pl.BlockSpec((_BLK, w.shape[1]), lambda i: (i, 0)),
        grid=(n // _BLK,),
    )(tab_s, x, w)


def _tc2_body(tab_s_ref, tab_d_ref, part_ref, b_ref, w_ref, o_ref):
    norm_in = _norm_from_tab(tab_d_ref[...])
    norm_out = _norm_from_tab(tab_s_ref[...])
    p = part_ref[...]
    h = jax.nn.relu((p[0] + p[1]) * norm_in[:, None] + b_ref[...])
    o_ref[...] = jnp.dot(h, w_ref[...],
                         preferred_element_type=jnp.float32) * norm_out[:, None]


def _tc2(tab_s, tab_d, part, b, w):
    n = part.shape[1]
    return pl.pallas_call(
        _tc2_body,
        out_shape=jax.ShapeDtypeStruct((n, w.shape[1]), jnp.float32),
        in_specs=[
            pl.BlockSpec((NC, _BLK, FEA), lambda i: (0, i, 0)),
            pl.BlockSpec((NC, _BLK, FEA), lambda i: (0, i, 0)),
            pl.BlockSpec((NC, _BLK, FEA), lambda i: (0, i, 0)),
            pl.BlockSpec((FEA,), lambda i: (0,)),
            pl.BlockSpec((FEA, w.shape[1]), lambda i: (0, 0)),
        ],
        out_specs=pl.BlockSpec((_BLK, w.shape[1]), lambda i: (i, 0)),
        grid=(n // _BLK,),
    )(tab_s, tab_d, part, b, w)


def _tc3_body(tab_d_ref, part_ref, b_ref, w_ref, bfc_ref, o_ref):
    norm_in = _norm_from_tab(tab_d_ref[...])
    p = part_ref[...]
    h = jax.nn.relu((p[0] + p[1]) * norm_in[:, None] + b_ref[...])
    o_ref[...] = jnp.dot(h, w_ref[...],
                         preferred_element_type=jnp.float32) + bfc_ref[...]


def _tc3(tab_d, part, b, w, bfc):
    n = part.shape[1]
    m = w.shape[1]
    return pl.pallas_call(
        _tc3_body,
        out_shape=jax.ShapeDtypeStruct((n, m), jnp.float32),
        in_specs=[
            pl.BlockSpec((NC, _BLK, FEA), lambda i: (0, i, 0)),
            pl.BlockSpec((NC, _BLK, FEA), lambda i: (0, i, 0)),
            pl.BlockSpec((FEA,), lambda i: (0,)),
            pl.BlockSpec((FEA, m), lambda i: (0, 0)),
            pl.BlockSpec((m,), lambda i: (0,)),
        ],
        out_specs=pl.BlockSpec((_BLK, m), lambda i: (i, 0)),
        grid=(n // _BLK,),
    )(tab_d, part, b, w, bfc)


def kernel(features, edge_index, W0, b0, W1, b1, Wfc, bfc):
    src = edge_index[0].astype(jnp.int32).reshape(NW, NCHUNK, CHUNK)
    dst = edge_index[1].astype(jnp.int32).reshape(NW, NCHUNK, CHUNK)
    ones128 = jnp.ones((CHUNK, FEA), jnp.float32)
    zeros128 = jnp.zeros((ROWS_PER_TILE, FEA), jnp.float32)

    tab_s, tab_d = _deg_kernel(src, dst, ones128, zeros128)
    tab_s = tab_s.reshape(NC, N_NODES, FEA)
    tab_d = tab_d.reshape(NC, N_NODES, FEA)

    ht0 = _tc1(tab_s, features, W0)
    part0 = _agg_kernel(ht0, src, dst, zeros128).reshape(NC, N_NODES, FEA)
    ht1 = _tc2(tab_s, tab_d, part0, b0, W1)
    part1 = _agg_kernel(ht1, src, dst, zeros128).reshape(NC, N_NODES, FEA)
    return _tc3(tab_d, part1, b1, Wfc, bfc)


# speedup vs baseline: 8.3559x; 1.3708x over previous
import functools

import jax
import jax.numpy as jnp
from jax import lax
from jax.experimental import pallas as pl
from jax.experimental.pallas import tpu as pltpu
from jax.experimental.pallas import tpu_sc as plsc

N_NODES = 10000
N_EDGES = 320000
FEA = 128
NC = 2
NS = 16
NW = NC * NS
EDGES_PER_TILE = N_EDGES // NW
CHUNK = 125
NCHUNK = EDGES_PER_TILE // CHUNK
PHASES = 2
PCHUNK = NCHUNK // PHASES
ZROW = 640
ZLAST = N_NODES - (NS - 1) * ZROW

_mesh = plsc.VectorSubcoreMesh(
    core_axis_name="c", subcore_axis_name="s", num_cores=NC, num_subcores=NS)


def _zero_slice(zeros_hbm, tab, s):
    @pl.when(s < NS - 1)
    def _():
        pltpu.sync_copy(zeros_hbm, tab.at[pl.ds(s * ZROW, ZROW)])

    @pl.when(s == NS - 1)
    def _():
        pltpu.sync_copy(zeros_hbm.at[pl.ds(0, ZLAST)],
                        tab.at[pl.ds((NS - 1) * ZROW, ZLAST)])


def _flush_slice(tab, out_hbm, c, s):
    @pl.when(s < NS - 1)
    def _():
        pltpu.sync_copy(tab.at[pl.ds(s * ZROW, ZROW)],
                        out_hbm.at[c, pl.ds(s * ZROW, ZROW)])

    @pl.when(s == NS - 1)
    def _():
        pltpu.sync_copy(tab.at[pl.ds((NS - 1) * ZROW, ZLAST)],
                        out_hbm.at[c, pl.ds((NS - 1) * ZROW, ZLAST)])


@functools.partial(
    pl.kernel,
    out_type=[jax.ShapeDtypeStruct((NC, N_NODES, FEA), jnp.float32),
              jax.ShapeDtypeStruct((NC, N_NODES, FEA), jnp.float32)],
    mesh=_mesh,
    scratch_types=[
        pltpu.VMEM((NCHUNK, CHUNK), jnp.int32),
        pltpu.VMEM((NCHUNK, CHUNK), jnp.int32),
        pltpu.VMEM((CHUNK, FEA), jnp.float32),
        pltpu.VMEM_SHARED((N_NODES, FEA), jnp.float32),
    ],
)
def _deg_kernel(src_hbm, dst_hbm, ones_hbm, zeros_hbm,
                out_s, out_d, src_v, dst_v, ones_v, tab):
    c = lax.axis_index("c")
    s = lax.axis_index("s")
    wid = c * NS + s
    pltpu.sync_copy(src_hbm.at[wid], src_v)
    pltpu.sync_copy(dst_hbm.at[wid], dst_v)
    pltpu.sync_copy(ones_hbm, ones_v)
    for idx_v, out_hbm in ((src_v, out_s), (dst_v, out_d)):
        _zero_slice(zeros_hbm, tab, s)
        plsc.subcore_barrier()

        def body(j, carry):
            pltpu.sync_copy(ones_v, tab.at[idx_v.at[j]], add=True)
            return carry

        lax.fori_loop(0, NCHUNK, body, 0)
        plsc.subcore_barrier()
        _flush_slice(tab, out_hbm, c, s)
        plsc.subcore_barrier()


@functools.partial(
    pl.kernel,
    out_type=jax.ShapeDtypeStruct((NC, N_NODES, FEA), jnp.float32),
    mesh=_mesh,
    scratch_types=[
        pltpu.VMEM((PCHUNK, CHUNK), jnp.int32),
        pltpu.VMEM((PCHUNK, CHUNK), jnp.int32),
        pltpu.VMEM((CHUNK, FEA), jnp.float32),
        pltpu.VMEM((CHUNK, FEA), jnp.float32),
        pltpu.VMEM_SHARED((N_NODES, FEA), jnp.float32),
        pltpu.SemaphoreType.DMA,
        pltpu.SemaphoreType.DMA,
    ],
)
def _agg_kernel(ht_hbm, src_hbm, dst_hbm, zeros_hbm,
                out_hbm, src_v, dst_v, rows0, rows1, accum, sem0, sem1):
    c = lax.axis_index("c")
    s = lax.axis_index("s")
    wid = c * NS + s
    _zero_slice(zeros_hbm, accum, s)
    plsc.subcore_barrier()

    for p in range(PHASES):
        pltpu.sync_copy(src_hbm.at[wid, pl.ds(p * PCHUNK, PCHUNK)], src_v)
        pltpu.sync_copy(dst_hbm.at[wid, pl.ds(p * PCHUNK, PCHUNK)], dst_v)
        pltpu.async_copy(ht_hbm.at[src_v.at[0]], rows0, sem0)

        def body(jj, carry):
            a = 2 * jj
            b = a + 1
            pltpu.async_copy(ht_hbm.at[src_v.at[b]], rows1, sem1)
            pltpu.make_async_copy(ht_hbm.at[src_v.at[a]], rows0, sem0).wait()
            pltpu.sync_copy(rows0, accum.at[dst_v.at[a]], add=True)

            @pl.when(jj < PCHUNK // 2 - 1)
            def _():
                pltpu.async_copy(ht_hbm.at[src_v.at[a + 2]], rows0, sem0)

            pltpu.make_async_copy(ht_hbm.at[src_v.at[b]], rows1, sem1).wait()
            pltpu.sync_copy(rows1, accum.at[dst_v.at[b]], add=True)
            return carry

        lax.fori_loop(0, PCHUNK // 2, body, 0)

    plsc.subcore_barrier()
    _flush_slice(accum, out_hbm, c, s)


_BLK = 1000


def _norm_from_tab(tab):
    deg = tab[0, :, 0] + tab[1, :, 0]
    return jnp.where(deg > 0, lax.rsqrt(deg), 0.0)


def _tc1_body(tab_s_ref, x_ref, w_ref, o_ref):
    norm_out = _norm_from_tab(tab_s_ref[...])
    o_ref[...] = jnp.dot(x_ref[...], w_ref[...],
                         preferred_element_type=jnp.float32) * norm_out[:, None]


def _tc1(tab_s, x, w):
    n, k = x.shape
    return pl.pallas_call(
        _tc1_body,
        out_shape=jax.ShapeDtypeStruct((n, w.shape[1]), jnp.float32),
        in_specs=[
            pl.BlockSpec((NC, _BLK, FEA), lambda i: (0, i, 0)),
            pl.BlockSpec((_BLK, k), lambda i: (i, 0)),
            pl.BlockSpec((k, w.shape[1]), lambda i: (0, 0)),
        ],
        out_specs=pl.BlockSpec((_BLK, w.shape[1]), lambda i: (i, 0)),
        grid=(n // _BLK,),
    )(tab_s, x, w)


def _tc2_body(tab_s_ref, tab_d_ref, part_ref, b_ref, w_ref, o_ref):
    norm_in = _norm_from_tab(tab_d_ref[...])
    norm_out = _norm_from_tab(tab_s_ref[...])
    p = part_ref[...]
    h = jax.nn.relu((p[0] + p[1]) * norm_in[:, None] + b_ref[...])
    o_ref[...] = jnp.dot(h, w_ref[...],
                         preferred_element_type=jnp.float32) * norm_out[:, None]


def _tc2(tab_s, tab_d, part, b, w):
    n = part.shape[1]
    return pl.pallas_call(
        _tc2_body,
        out_shape=jax.ShapeDtypeStruct((n, w.shape[1]), jnp.float32),
        in_specs=[
            pl.BlockSpec((NC, _BLK, FEA), lambda i: (0, i, 0)),
            pl.BlockSpec((NC, _BLK, FEA), lambda i: (0, i, 0)),
            pl.BlockSpec((NC, _BLK, FEA), lambda i: (0, i, 0)),
            pl.BlockSpec((FEA,), lambda i: (0,)),
            pl.BlockSpec((FEA, w.shape[1]), lambda i: (0, 0)),
        ],
        out_specs=pl.BlockSpec((_BLK, w.shape[1]), lambda i: (i, 0)),
        grid=(n // _BLK,),
    )(tab_s, tab_d, part, b, w)


def _tc3_body(tab_d_ref, part_ref, b_ref, w_ref, bfc_ref, o_ref):
    norm_in = _norm_from_tab(tab_d_ref[...])
    p = part_ref[...]
    h = jax.nn.relu((p[0] + p[1]) * norm_in[:, None] + b_ref[...])
    o_ref[...] = jnp.dot(h, w_ref[...],
                         preferred_element_type=jnp.float32) + bfc_ref[...]


def _tc3(tab_d, part, b, w, bfc):
    n = part.shape[1]
    m = w.shape[1]
    return pl.pallas_call(
        _tc3_body,
        out_shape=jax.ShapeDtypeStruct((n, m), jnp.float32),
        in_specs=[
            pl.BlockSpec((NC, _BLK, FEA), lambda i: (0, i, 0)),
            pl.BlockSpec((NC, _BLK, FEA), lambda i: (0, i, 0)),
            pl.BlockSpec((FEA,), lambda i: (0,)),
            pl.BlockSpec((FEA, m), lambda i: (0, 0)),
            pl.BlockSpec((m,), lambda i: (0,)),
        ],
        out_specs=pl.BlockSpec((_BLK, m), lambda i: (i, 0)),
        grid=(n // _BLK,),
    )(tab_d, part, b, w, bfc)


def kernel(features, edge_index, W0, b0, W1, b1, Wfc, bfc):
    src = edge_index[0].astype(jnp.int32).reshape(NW, NCHUNK, CHUNK)
    dst = edge_index[1].astype(jnp.int32).reshape(NW, NCHUNK, CHUNK)
    ones128 = jnp.ones((CHUNK, FEA), jnp.float32)
    zeros128 = jnp.zeros((ZROW, FEA), jnp.float32)

    tab_s, tab_d = _deg_kernel(src, dst, ones128, zeros128)

    ht0 = _tc1(tab_s, features, W0)
    part0 = _agg_kernel(ht0, src, dst, zeros128)
    ht1 = _tc2(tab_s, tab_d, part0, b0, W1)
    part1 = _agg_kernel(ht1, src, dst, zeros128)
    return _tc3(tab_d, part1, b1, Wfc, bfc)


# speedup vs baseline: 8.3844x; 1.0034x over previous
import functools

import jax
import jax.numpy as jnp
from jax import lax
from jax.experimental import pallas as pl
from jax.experimental.pallas import tpu as pltpu
from jax.experimental.pallas import tpu_sc as plsc

N_NODES = 10000
N_EDGES = 320000
FEA = 128
NC = 2
NS = 16
NW = NC * NS
EDGES_PER_TILE = N_EDGES // NW
CHUNK = 125
NCHUNK = EDGES_PER_TILE // CHUNK
PHASES = 2
PCHUNK = NCHUNK // PHASES
ZROW = 640
ZLAST = N_NODES - (NS - 1) * ZROW

_mesh = plsc.VectorSubcoreMesh(
    core_axis_name="c", subcore_axis_name="s", num_cores=NC, num_subcores=NS)


def _zero_slice(zeros_hbm, tab, s):
    @pl.when(s < NS - 1)
    def _():
        pltpu.sync_copy(zeros_hbm, tab.at[pl.ds(s * ZROW, ZROW)])

    @pl.when(s == NS - 1)
    def _():
        pltpu.sync_copy(zeros_hbm.at[pl.ds(0, ZLAST)],
                        tab.at[pl.ds((NS - 1) * ZROW, ZLAST)])


def _flush_slice(tab, out_hbm, c, s):
    @pl.when(s < NS - 1)
    def _():
        pltpu.sync_copy(tab.at[pl.ds(s * ZROW, ZROW)],
                        out_hbm.at[c, pl.ds(s * ZROW, ZROW)])

    @pl.when(s == NS - 1)
    def _():
        pltpu.sync_copy(tab.at[pl.ds((NS - 1) * ZROW, ZLAST)],
                        out_hbm.at[c, pl.ds((NS - 1) * ZROW, ZLAST)])


@functools.partial(
    pl.kernel,
    out_type=[jax.ShapeDtypeStruct((NC, N_NODES, FEA), jnp.float32),
              jax.ShapeDtypeStruct((NC, N_NODES, FEA), jnp.float32)],
    mesh=_mesh,
    scratch_types=[
        pltpu.VMEM((NCHUNK, CHUNK), jnp.int32),
        pltpu.VMEM((NCHUNK, CHUNK), jnp.int32),
        pltpu.VMEM((CHUNK, FEA), jnp.float32),
        pltpu.VMEM_SHARED((N_NODES, FEA), jnp.float32),
        pltpu.SemaphoreType.DMA,
    ],
)
def _deg_kernel(src_hbm, dst_hbm, ones_hbm, zeros_hbm,
                out_s, out_d, src_v, dst_v, ones_v, tab, sem):
    c = lax.axis_index("c")
    s = lax.axis_index("s")
    wid = c * NS + s
    pltpu.sync_copy(src_hbm.at[wid], src_v)
    pltpu.sync_copy(dst_hbm.at[wid], dst_v)
    pltpu.sync_copy(ones_hbm, ones_v)
    for idx_v, out_hbm in ((src_v, out_s), (dst_v, out_d)):
        _zero_slice(zeros_hbm, tab, s)
        plsc.subcore_barrier()

        def body(j, carry):
            pltpu.async_copy(ones_v, tab.at[idx_v.at[j]], sem, add=True)
            return carry

        def drain(j, carry):
            pltpu.make_async_copy(ones_v, tab.at[idx_v.at[0]], sem).wait()
            return carry

        lax.fori_loop(0, NCHUNK, body, 0)
        lax.fori_loop(0, NCHUNK, drain, 0)
        plsc.subcore_barrier()
        _flush_slice(tab, out_hbm, c, s)
        plsc.subcore_barrier()


@functools.partial(
    pl.kernel,
    out_type=jax.ShapeDtypeStruct((NC, N_NODES, FEA), jnp.float32),
    mesh=_mesh,
    scratch_types=[
        pltpu.VMEM((PCHUNK, CHUNK), jnp.int32),
        pltpu.VMEM((PCHUNK, CHUNK), jnp.int32),
        pltpu.VMEM((CHUNK, FEA), jnp.float32),
        pltpu.VMEM((CHUNK, FEA), jnp.float32),
        pltpu.VMEM_SHARED((N_NODES, FEA), jnp.float32),
        pltpu.SemaphoreType.DMA,
        pltpu.SemaphoreType.DMA,
    ],
)
def _agg_kernel(ht_hbm, src_hbm, dst_hbm, zeros_hbm,
                out_hbm, src_v, dst_v, rows0, rows1, accum, sem0, sem1):
    c = lax.axis_index("c")
    s = lax.axis_index("s")
    wid = c * NS + s
    _zero_slice(zeros_hbm, accum, s)
    plsc.subcore_barrier()

    for p in range(PHASES):
        pltpu.sync_copy(src_hbm.at[wid, pl.ds(p * PCHUNK, PCHUNK)], src_v)
        pltpu.sync_copy(dst_hbm.at[wid, pl.ds(p * PCHUNK, PCHUNK)], dst_v)
        pltpu.async_copy(ht_hbm.at[src_v.at[0]], rows0, sem0)

        def body(jj, carry):
            a = 2 * jj
            b = a + 1
            pltpu.async_copy(ht_hbm.at[src_v.at[b]], rows1, sem1)
            pltpu.make_async_copy(ht_hbm.at[src_v.at[a]], rows0, sem0).wait()
            pltpu.sync_copy(rows0, accum.at[dst_v.at[a]], add=True)

            @pl.when(jj < PCHUNK // 2 - 1)
            def _():
                pltpu.async_copy(ht_hbm.at[src_v.at[a + 2]], rows0, sem0)

            pltpu.make_async_copy(ht_hbm.at[src_v.at[b]], rows1, sem1).wait()
            pltpu.sync_copy(rows1, accum.at[dst_v.at[b]], add=True)
            return carry

        lax.fori_loop(0, PCHUNK // 2, body, 0)

    plsc.subcore_barrier()
    _flush_slice(accum, out_hbm, c, s)


_BLK = 1000


def _norm_from_tab(tab):
    deg = tab[0, :, 0] + tab[1, :, 0]
    return jnp.where(deg > 0, lax.rsqrt(deg), 0.0)


def _tc1_body(tab_s_ref, x_ref, w_ref, o_ref):
    norm_out = _norm_from_tab(tab_s_ref[...])
    o_ref[...] = jnp.dot(x_ref[...], w_ref[...],
                         preferred_element_type=jnp.float32) * norm_out[:, None]


def _tc1(tab_s, x, w):
    n, k = x.shape
    return pl.pallas_call(
        _tc1_body,
        out_shape=jax.ShapeDtypeStruct((n, w.shape[1]), jnp.float32),
        in_specs=[
            pl.BlockSpec((NC, _BLK, FEA), lambda i: (0, i, 0)),
            pl.BlockSpec((_BLK, k), lambda i: (i, 0)),
            pl.BlockSpec((k, w.shape[1]), lambda i: (0, 0)),
        ],
        out_specs=pl.BlockSpec((_BLK, w.shape[1]), lambda i: (i, 0)),
        grid=(n // _BLK,),
    )(tab_s, x, w)


def _tc2_body(tab_s_ref, tab_d_ref, part_ref, b_ref, w_ref, o_ref):
    norm_in = _norm_from_tab(tab_d_ref[...])
    norm_out = _norm_from_tab(tab_s_ref[...])
    p = part_ref[...]
    h = jax.nn.relu((p[0] + p[1]) * norm_in[:, None] + b_ref[...])
    o_ref[...] = jnp.dot(h, w_ref[...],
                         preferred_element_type=jnp.float32) * norm_out[:, None]


def _tc2(tab_s, tab_d, part, b, w):
    n = part.shape[1]
    return pl.pallas_call(
        _tc2_body,
        out_shape=jax.ShapeDtypeStruct((n, w.shape[1]), jnp.float32),
        in_specs=[
            pl.BlockSpec((NC, _BLK, FEA), lambda i: (0, i, 0)),
            pl.BlockSpec((NC, _BLK, FEA), lambda i: (0, i, 0)),
            pl.BlockSpec((NC, _BLK, FEA), lambda i: (0, i, 0)),
            pl.BlockSpec((FEA,), lambda i: (0,)),
            pl.BlockSpec((FEA, w.shape[1]), lambda i: (0, 0)),
        ],
        out_specs=pl.BlockSpec((_BLK, w.shape[1]), lambda i: (i, 0)),
        grid=(n // _BLK,),
    )(tab_s, tab_d, part, b, w)


def _tc3_body(tab_d_ref, part_ref, b_ref, w_ref, bfc_ref, o_ref):
    norm_in = _norm_from_tab(tab_d_ref[...])
    p = part_ref[...]
    h = jax.nn.relu((p[0] + p[1]) * norm_in[:, None] + b_ref[...])
    o_ref[...] = jnp.dot(h, w_ref[...],
                         preferred_element_type=jnp.float32) + bfc_ref[...]


def _tc3(tab_d, part, b, w, bfc):
    n = part.shape[1]
    m = w.shape[1]
    return pl.pallas_call(
        _tc3_body,
        out_shape=jax.ShapeDtypeStruct((n, m), jnp.float32),
        in_specs=[
            pl.BlockSpec((NC, _BLK, FEA), lambda i: (0, i, 0)),
            pl.BlockSpec((NC, _BLK, FEA), lambda i: (0, i, 0)),
            pl.BlockSpec((FEA,), lambda i: (0,)),
            pl.BlockSpec((FEA, m), lambda i: (0, 0)),
            pl.BlockSpec((m,), lambda i: (0,)),
        ],
        out_specs=pl.BlockSpec((_BLK, m), lambda i: (i, 0)),
        grid=(n // _BLK,),
    )(tab_d, part, b, w, bfc)


def kernel(features, edge_index, W0, b0, W1, b1, Wfc, bfc):
    src = edge_index[0].astype(jnp.int32).reshape(NW, NCHUNK, CHUNK)
    dst = edge_index[1].astype(jnp.int32).reshape(NW, NCHUNK, CHUNK)
    ones128 = jnp.ones((CHUNK, FEA), jnp.float32)
    zeros128 = jnp.zeros((ZROW, FEA), jnp.float32)

    tab_s, tab_d = _deg_kernel(src, dst, ones128, zeros128)

    ht0 = _tc1(tab_s, features, W0)
    part0 = _agg_kernel(ht0, src, dst, zeros128)
    ht1 = _tc2(tab_s, tab_d, part0, b0, W1)
    part1 = _agg_kernel(ht1, src, dst, zeros128)
    return _tc3(tab_d, part1, b1, Wfc, bfc)


# speedup vs baseline: 8.6127x; 1.0272x over previous
import functools

import jax
import jax.numpy as jnp
from jax import lax
from jax.experimental import pallas as pl
from jax.experimental.pallas import tpu as pltpu
from jax.experimental.pallas import tpu_sc as plsc

N_NODES = 10000
N_EDGES = 320000
FEA = 128
NC = 2
NS = 16
NW = NC * NS
EDGES_PER_TILE = N_EDGES // NW
CHUNK = 125
NCHUNK = EDGES_PER_TILE // CHUNK
PHASES = 2
PCHUNK = NCHUNK // PHASES
ZROW = 640
ZLAST = N_NODES - (NS - 1) * ZROW

_mesh = plsc.VectorSubcoreMesh(
    core_axis_name="c", subcore_axis_name="s", num_cores=NC, num_subcores=NS)


def _zero_slice(zeros_hbm, tab, s):
    @pl.when(s < NS - 1)
    def _():
        pltpu.sync_copy(zeros_hbm, tab.at[pl.ds(s * ZROW, ZROW)])

    @pl.when(s == NS - 1)
    def _():
        pltpu.sync_copy(zeros_hbm.at[pl.ds(0, ZLAST)],
                        tab.at[pl.ds((NS - 1) * ZROW, ZLAST)])


def _flush_slice(tab, out_hbm, c, s):
    @pl.when(s < NS - 1)
    def _():
        pltpu.sync_copy(tab.at[pl.ds(s * ZROW, ZROW)],
                        out_hbm.at[c, pl.ds(s * ZROW, ZROW)])

    @pl.when(s == NS - 1)
    def _():
        pltpu.sync_copy(tab.at[pl.ds((NS - 1) * ZROW, ZLAST)],
                        out_hbm.at[c, pl.ds((NS - 1) * ZROW, ZLAST)])


@functools.partial(
    pl.kernel,
    out_type=[jax.ShapeDtypeStruct((NC, N_NODES, FEA), jnp.float32),
              jax.ShapeDtypeStruct((NC, N_NODES, FEA), jnp.float32)],
    mesh=_mesh,
    scratch_types=[
        pltpu.VMEM((NCHUNK, CHUNK), jnp.int32),
        pltpu.VMEM((NCHUNK, CHUNK), jnp.int32),
        pltpu.VMEM((CHUNK, FEA), jnp.float32),
        pltpu.VMEM_SHARED((N_NODES, FEA), jnp.float32),
        pltpu.SemaphoreType.DMA,
    ],
)
def _deg_kernel(edge_hbm, ones_hbm, zeros_hbm,
                out_s, out_d, src_v, dst_v, ones_v, tab, sem):
    c = lax.axis_index("c")
    s = lax.axis_index("s")
    wid = c * NS + s
    pltpu.sync_copy(edge_hbm.at[0, wid], src_v)
    pltpu.sync_copy(edge_hbm.at[1, wid], dst_v)
    pltpu.sync_copy(ones_hbm, ones_v)
    for idx_v, out_hbm in ((src_v, out_s), (dst_v, out_d)):
        _zero_slice(zeros_hbm, tab, s)
        plsc.subcore_barrier()

        def body(j, carry):
            pltpu.async_copy(ones_v, tab.at[idx_v.at[j]], sem, add=True)
            return carry

        def drain(j, carry):
            pltpu.make_async_copy(ones_v, tab.at[idx_v.at[0]], sem).wait()
            return carry

        lax.fori_loop(0, NCHUNK, body, 0)
        lax.fori_loop(0, NCHUNK, drain, 0)
        plsc.subcore_barrier()
        _flush_slice(tab, out_hbm, c, s)
        plsc.subcore_barrier()


@functools.partial(
    pl.kernel,
    out_type=jax.ShapeDtypeStruct((NC, N_NODES, FEA), jnp.float32),
    mesh=_mesh,
    scratch_types=[
        pltpu.VMEM((PCHUNK, CHUNK), jnp.int32),
        pltpu.VMEM((PCHUNK, CHUNK), jnp.int32),
        pltpu.VMEM((CHUNK, FEA), jnp.float32),
        pltpu.VMEM((CHUNK, FEA), jnp.float32),
        pltpu.VMEM_SHARED((N_NODES, FEA), jnp.float32),
        pltpu.SemaphoreType.DMA,
        pltpu.SemaphoreType.DMA,
    ],
)
def _agg_kernel(ht_hbm, edge_hbm, zeros_hbm,
                out_hbm, src_v, dst_v, rows0, rows1, accum, sem0, sem1):
    c = lax.axis_index("c")
    s = lax.axis_index("s")
    wid = c * NS + s
    _zero_slice(zeros_hbm, accum, s)
    plsc.subcore_barrier()

    for p in range(PHASES):
        pltpu.sync_copy(edge_hbm.at[0, wid, pl.ds(p * PCHUNK, PCHUNK)], src_v)
        pltpu.sync_copy(edge_hbm.at[1, wid, pl.ds(p * PCHUNK, PCHUNK)], dst_v)
        pltpu.async_copy(ht_hbm.at[src_v.at[0]], rows0, sem0)

        def body(jj, carry):
            a = 2 * jj
            b = a + 1
            pltpu.async_copy(ht_hbm.at[src_v.at[b]], rows1, sem1)
            pltpu.make_async_copy(ht_hbm.at[src_v.at[a]], rows0, sem0).wait()
            pltpu.sync_copy(rows0, accum.at[dst_v.at[a]], add=True)

            @pl.when(jj < PCHUNK // 2 - 1)
            def _():
                pltpu.async_copy(ht_hbm.at[src_v.at[a + 2]], rows0, sem0)

            pltpu.make_async_copy(ht_hbm.at[src_v.at[b]], rows1, sem1).wait()
            pltpu.sync_copy(rows1, accum.at[dst_v.at[b]], add=True)
            return carry

        lax.fori_loop(0, PCHUNK // 2, body, 0)

    plsc.subcore_barrier()
    _flush_slice(accum, out_hbm, c, s)


_BLK = 1000


def _norm_from_tab(tab):
    deg = tab[0, :, 0] + tab[1, :, 0]
    return jnp.where(deg > 0, lax.rsqrt(deg), 0.0)


def _tc1_body(tab_s_ref, x_ref, w_ref, o_ref):
    norm_out = _norm_from_tab(tab_s_ref[...])
    o_ref[...] = jnp.dot(x_ref[...], w_ref[...],
                         preferred_element_type=jnp.float32) * norm_out[:, None]


def _tc1(tab_s, x, w):
    n, k = x.shape
    return pl.pallas_call(
        _tc1_body,
        out_shape=jax.ShapeDtypeStruct((n, w.shape[1]), jnp.float32),
        in_specs=[
            pl.BlockSpec((NC, _BLK, FEA), lambda i: (0, i, 0)),
            pl.BlockSpec((_BLK, k), lambda i: (i, 0)),
            pl.BlockSpec((k, w.shape[1]), lambda i: (0, 0)),
        ],
        out_specs=pl.BlockSpec((_BLK, w.shape[1]), lambda i: (i, 0)),
        grid=(n // _BLK,),
    )(tab_s, x, w)


def _tc2_body(tab_s_ref, tab_d_ref, part_ref, b_ref, w_ref, o_ref):
    norm_in = _norm_from_tab(tab_d_ref[...])
    norm_out = _norm_from_tab(tab_s_ref[...])
    p = part_ref[...]
    h = jax.nn.relu((p[0] + p[1]) * norm_in[:, None] + b_ref[...])
    o_ref[...] = jnp.dot(h, w_ref[...],
                         preferred_element_type=jnp.float32) * norm_out[:, None]


def _tc2(tab_s, tab_d, part, b, w):
    n = part.shape[1]
    return pl.pallas_call(
        _tc2_body,
        out_shape=jax.ShapeDtypeStruct((n, w.shape[1]), jnp.float32),
        in_specs=[
            pl.BlockSpec((NC, _BLK, FEA), lambda i: (0, i, 0)),
            pl.BlockSpec((NC, _BLK, FEA), lambda i: (0, i, 0)),
            pl.BlockSpec((NC, _BLK, FEA), lambda i: (0, i, 0)),
            pl.BlockSpec((FEA,), lambda i: (0,)),
            pl.BlockSpec((FEA, w.shape[1]), lambda i: (0, 0)),
        ],
        out_specs=pl.BlockSpec((_BLK, w.shape[1]), lambda i: (i, 0)),
        grid=(n // _BLK,),
    )(tab_s, tab_d, part, b, w)


def _tc3_body(tab_d_ref, part_ref, b_ref, w_ref, bfc_ref, o_ref):
    norm_in = _norm_from_tab(tab_d_ref[...])
    p = part_ref[...]
    h = jax.nn.relu((p[0] + p[1]) * norm_in[:, None] + b_ref[...])
    o_ref[...] = jnp.dot(h, w_ref[...],
                         preferred_element_type=jnp.float32) + bfc_ref[...]


def _tc3(tab_d, part, b, w, bfc):
    n = part.shape[1]
    m = w.shape[1]
    return pl.pallas_call(
        _tc3_body,
        out_shape=jax.ShapeDtypeStruct((n, m), jnp.float32),
        in_specs=[
            pl.BlockSpec((NC, _BLK, FEA), lambda i: (0, i, 0)),
            pl.BlockSpec((NC, _BLK, FEA), lambda i: (0, i, 0)),
            pl.BlockSpec((FEA,), lambda i: (0,)),
            pl.BlockSpec((FEA, m), lambda i: (0, 0)),
            pl.BlockSpec((m,), lambda i: (0,)),
        ],
        out_specs=pl.BlockSpec((_BLK, m), lambda i: (i, 0)),
        grid=(n // _BLK,),
    )(tab_d, part, b, w, bfc)


def kernel(features, edge_index, W0, b0, W1, b1, Wfc, bfc):
    edges = edge_index.astype(jnp.int32).reshape(2, NW, NCHUNK, CHUNK)
    ones128 = jnp.ones((CHUNK, FEA), jnp.float32)
    zeros128 = jnp.zeros((ZROW, FEA), jnp.float32)

    tab_s, tab_d = _deg_kernel(edges, ones128, zeros128)

    ht0 = _tc1(tab_s, features, W0)
    part0 = _agg_kernel(ht0, edges, zeros128)
    ht1 = _tc2(tab_s, tab_d, part0, b0, W1)
    part1 = _agg_kernel(ht1, edges, zeros128)
    return _tc3(tab_d, part1, b1, Wfc, bfc)


# speedup vs baseline: 9.0084x; 1.0460x over previous
import functools

import jax
import jax.numpy as jnp
from jax import lax
from jax.experimental import pallas as pl
from jax.experimental.pallas import tpu as pltpu
from jax.experimental.pallas import tpu_sc as plsc

N_NODES = 10000
N_EDGES = 320000
FEA = 128
NC = 2
NS = 16
NW = NC * NS
EDGES_PER_TILE = N_EDGES // NW
CHUNK = 125
NCHUNK = EDGES_PER_TILE // CHUNK
PHASES = 2
PCHUNK = NCHUNK // PHASES
ZROW = 640
ZLAST = N_NODES - (NS - 1) * ZROW

_mesh = plsc.VectorSubcoreMesh(
    core_axis_name="c", subcore_axis_name="s", num_cores=NC, num_subcores=NS)


def _zero_slice(zeros_hbm, tab, s):
    @pl.when(s < NS - 1)
    def _():
        pltpu.sync_copy(zeros_hbm, tab.at[pl.ds(s * ZROW, ZROW)])

    @pl.when(s == NS - 1)
    def _():
        pltpu.sync_copy(zeros_hbm.at[pl.ds(0, ZLAST)],
                        tab.at[pl.ds((NS - 1) * ZROW, ZLAST)])


def _flush_slice(tab, out_hbm, c, s):
    @pl.when(s < NS - 1)
    def _():
        pltpu.sync_copy(tab.at[pl.ds(s * ZROW, ZROW)],
                        out_hbm.at[c, pl.ds(s * ZROW, ZROW)])

    @pl.when(s == NS - 1)
    def _():
        pltpu.sync_copy(tab.at[pl.ds((NS - 1) * ZROW, ZLAST)],
                        out_hbm.at[c, pl.ds((NS - 1) * ZROW, ZLAST)])


@functools.partial(
    pl.kernel,
    out_type=jax.ShapeDtypeStruct((2, N_NODES, FEA), jnp.float32),
    mesh=_mesh,
    scratch_types=[
        pltpu.VMEM((2 * NCHUNK, CHUNK), jnp.int32),
        pltpu.VMEM((CHUNK, FEA), jnp.float32),
        pltpu.VMEM_SHARED((N_NODES, FEA), jnp.float32),
        pltpu.SemaphoreType.DMA,
    ],
)
def _deg_kernel(edge_hbm, ones_hbm, zeros_hbm, out, idx_v, ones_v, tab, sem):
    c = lax.axis_index("c")
    s = lax.axis_index("s")
    pltpu.sync_copy(edge_hbm.at[c, 2 * s], idx_v.at[pl.ds(0, NCHUNK)])
    pltpu.sync_copy(edge_hbm.at[c, 2 * s + 1], idx_v.at[pl.ds(NCHUNK, NCHUNK)])
    pltpu.sync_copy(ones_hbm, ones_v)
    _zero_slice(zeros_hbm, tab, s)
    plsc.subcore_barrier()

    def body(j, carry):
        pltpu.async_copy(ones_v, tab.at[idx_v.at[j]], sem, add=True)
        return carry

    def drain(j, carry):
        pltpu.make_async_copy(ones_v, tab.at[idx_v.at[0]], sem).wait()
        return carry

    lax.fori_loop(0, 2 * NCHUNK, body, 0)
    lax.fori_loop(0, 2 * NCHUNK, drain, 0)
    plsc.subcore_barrier()
    _flush_slice(tab, out, c, s)


@functools.partial(
    pl.kernel,
    out_type=jax.ShapeDtypeStruct((NC, N_NODES, FEA), jnp.float32),
    mesh=_mesh,
    scratch_types=[
        pltpu.VMEM((PCHUNK, CHUNK), jnp.int32),
        pltpu.VMEM((PCHUNK, CHUNK), jnp.int32),
        pltpu.VMEM((CHUNK, FEA), jnp.float32),
        pltpu.VMEM((CHUNK, FEA), jnp.float32),
        pltpu.VMEM_SHARED((N_NODES, FEA), jnp.float32),
        pltpu.SemaphoreType.DMA,
        pltpu.SemaphoreType.DMA,
    ],
)
def _agg_kernel(ht_hbm, edge_hbm, zeros_hbm,
                out_hbm, src_v, dst_v, rows0, rows1, accum, sem0, sem1):
    c = lax.axis_index("c")
    s = lax.axis_index("s")
    wid = c * NS + s
    _zero_slice(zeros_hbm, accum, s)
    plsc.subcore_barrier()

    for p in range(PHASES):
        pltpu.sync_copy(edge_hbm.at[0, wid, pl.ds(p * PCHUNK, PCHUNK)], src_v)
        pltpu.sync_copy(edge_hbm.at[1, wid, pl.ds(p * PCHUNK, PCHUNK)], dst_v)
        pltpu.async_copy(ht_hbm.at[src_v.at[0]], rows0, sem0)

        def body(jj, carry):
            a = 2 * jj
            b = a + 1
            pltpu.async_copy(ht_hbm.at[src_v.at[b]], rows1, sem1)
            pltpu.make_async_copy(ht_hbm.at[src_v.at[a]], rows0, sem0).wait()
            pltpu.sync_copy(rows0, accum.at[dst_v.at[a]], add=True)

            @pl.when(jj < PCHUNK // 2 - 1)
            def _():
                pltpu.async_copy(ht_hbm.at[src_v.at[a + 2]], rows0, sem0)

            pltpu.make_async_copy(ht_hbm.at[src_v.at[b]], rows1, sem1).wait()
            pltpu.sync_copy(rows1, accum.at[dst_v.at[b]], add=True)
            return carry

        lax.fori_loop(0, PCHUNK // 2, body, 0)

    plsc.subcore_barrier()
    _flush_slice(accum, out_hbm, c, s)


_BLK = 1000


def _norm_from_tab(tab):
    deg = tab[0, :, 0]
    return jnp.where(deg > 0, lax.rsqrt(deg), 0.0)


def _tc1_body(tab_s_ref, x_ref, w_ref, o_ref):
    norm_out = _norm_from_tab(tab_s_ref[...])
    o_ref[...] = jnp.dot(x_ref[...], w_ref[...],
                         preferred_element_type=jnp.float32) * norm_out[:, None]


def _tc1(tabs, x, w):
    n, k = x.shape
    return pl.pallas_call(
        _tc1_body,
        out_shape=jax.ShapeDtypeStruct((n, w.shape[1]), jnp.float32),
        in_specs=[
            pl.BlockSpec((1, _BLK, FEA), lambda i: (0, i, 0)),
            pl.BlockSpec((_BLK, k), lambda i: (i, 0)),
            pl.BlockSpec((k, w.shape[1]), lambda i: (0, 0)),
        ],
        out_specs=pl.BlockSpec((_BLK, w.shape[1]), lambda i: (i, 0)),
        grid=(n // _BLK,),
    )(tabs, x, w)


def _tc2_body(tab_s_ref, tab_d_ref, part_ref, b_ref, w_ref, o_ref):
    norm_in = _norm_from_tab(tab_d_ref[...])
    norm_out = _norm_from_tab(tab_s_ref[...])
    p = part_ref[...]
    h = jax.nn.relu((p[0] + p[1]) * norm_in[:, None] + b_ref[...])
    o_ref[...] = jnp.dot(h, w_ref[...],
                         preferred_element_type=jnp.float32) * norm_out[:, None]


def _tc2(tabs, part, b, w):
    n = part.shape[1]
    return pl.pallas_call(
        _tc2_body,
        out_shape=jax.ShapeDtypeStruct((n, w.shape[1]), jnp.float32),
        in_specs=[
            pl.BlockSpec((1, _BLK, FEA), lambda i: (0, i, 0)),
            pl.BlockSpec((1, _BLK, FEA), lambda i: (1, i, 0)),
            pl.BlockSpec((NC, _BLK, FEA), lambda i: (0, i, 0)),
            pl.BlockSpec((FEA,), lambda i: (0,)),
            pl.BlockSpec((FEA, w.shape[1]), lambda i: (0, 0)),
        ],
        out_specs=pl.BlockSpec((_BLK, w.shape[1]), lambda i: (i, 0)),
        grid=(n // _BLK,),
    )(tabs, tabs, part, b, w)


def _tc3_body(tab_d_ref, part_ref, b_ref, w_ref, bfc_ref, o_ref):
    norm_in = _norm_from_tab(tab_d_ref[...])
    p = part_ref[...]
    h = jax.nn.relu((p[0] + p[1]) * norm_in[:, None] + b_ref[...])
    o_ref[...] = jnp.dot(h, w_ref[...],
                         preferred_element_type=jnp.float32) + bfc_ref[...]


def _tc3(tabs, part, b, w, bfc):
    n = part.shape[1]
    m = w.shape[1]
    return pl.pallas_call(
        _tc3_body,
        out_shape=jax.ShapeDtypeStruct((n, m), jnp.float32),
        in_specs=[
            pl.BlockSpec((1, _BLK, FEA), lambda i: (1, i, 0)),
            pl.BlockSpec((NC, _BLK, FEA), lambda i: (0, i, 0)),
            pl.BlockSpec((FEA,), lambda i: (0,)),
            pl.BlockSpec((FEA, m), lambda i: (0, 0)),
            pl.BlockSpec((m,), lambda i: (0,)),
        ],
        out_specs=pl.BlockSpec((_BLK, m), lambda i: (i, 0)),
        grid=(n // _BLK,),
    )(tabs, part, b, w, bfc)


def kernel(features, edge_index, W0, b0, W1, b1, Wfc, bfc):
    edges = edge_index.astype(jnp.int32).reshape(2, NW, NCHUNK, CHUNK)
    ones128 = jnp.ones((CHUNK, FEA), jnp.float32)
    zeros128 = jnp.zeros((ZROW, FEA), jnp.float32)

    tabs = _deg_kernel(edges, ones128, zeros128)

    ht0 = _tc1(tabs, features, W0)
    part0 = _agg_kernel(ht0, edges, zeros128)
    ht1 = _tc2(tabs, part0, b0, W1)
    part1 = _agg_kernel(ht1, edges, zeros128)
    return _tc3(tabs, part1, b1, Wfc, bfc)
